# baseline (speedup 1.0000x reference)
import functools

import jax
import jax.numpy as jnp
from jax import lax
from jax.experimental import pallas as pl
from jax.experimental.pallas import tpu as pltpu
from jax.experimental.pallas import tpu_sc as plsc

B = 16384
L = 50
D = 32
NC = 2
NS = 16
NW = NC * NS
LANES = 16

BAGS_PER_STEP = 2
IDX_PER_STEP = BAGS_PER_STEP * L
BAGS_PER_W = B // NW
STEPS = BAGS_PER_W // BAGS_PER_STEP
NBUF = 2


@functools.cache
def _build():
    mesh = plsc.VectorSubcoreMesh(
        core_axis_name="c", subcore_axis_name="s", num_cores=NC, num_subcores=NS
    )

    @functools.partial(
        pl.kernel,
        out_type=jax.ShapeDtypeStruct((B, D), jnp.float32),
        mesh=mesh,
        compiler_params=pltpu.CompilerParams(use_tc_tiling_on_sc=False),
        scratch_types=[
            pltpu.VMEM((STEPS, IDX_PER_STEP), jnp.int32),
            pltpu.VMEM((NBUF, IDX_PER_STEP, D), jnp.float32),
            pltpu.VMEM((BAGS_PER_W, D), jnp.float32),
            pltpu.SemaphoreType.DMA((NBUF,)),
        ],
    )
    def embed_bag(idx_hbm, table_hbm, out_hbm, idx_v, rows_v, out_v, sems):
        wid = lax.axis_index("s") * NC + lax.axis_index("c")

        pltpu.sync_copy(idx_hbm.at[pl.ds(wid * STEPS, STEPS)], idx_v)

        def gather(step, slot):
            return pltpu.async_copy(
                table_hbm.at[idx_v.at[step]], rows_v.at[slot], sems.at[slot]
            )

        for b in range(NBUF):
            gather(b, b)

        inv = jnp.float32(1.0 / L)

        def outer(i, carry):
            gbase = i * NBUF
            for b in range(NBUF):
                g = gbase + b
                pltpu.make_async_copy(
                    table_hbm.at[idx_v.at[g]], rows_v.at[b], sems.at[b]
                ).wait()
                for bag in range(BAGS_PER_STEP):
                    r0 = bag * L
                    acc = [rows_v[b, r0 + j, pl.ds(h * LANES, LANES)]
                           for j in range(4) for h in range(2)]
                    for r in range(4, L):
                        acc[2 * (r % 4)] += rows_v[b, r0 + r, pl.ds(0, LANES)]
                        acc[2 * (r % 4) + 1] += rows_v[
                            b, r0 + r, pl.ds(LANES, LANES)]
                    out_row = g * BAGS_PER_STEP + bag
                    out_v[out_row, pl.ds(0, LANES)] = (
                        (acc[0] + acc[2]) + (acc[4] + acc[6])) * inv
                    out_v[out_row, pl.ds(LANES, LANES)] = (
                        (acc[1] + acc[3]) + (acc[5] + acc[7])) * inv

                @pl.when(g + NBUF < STEPS)
                def _():
                    gather(g + NBUF, b)

            return carry

        lax.fori_loop(0, STEPS // NBUF, outer, 0)

        pltpu.sync_copy(out_v, out_hbm.at[pl.ds(wid * BAGS_PER_W, BAGS_PER_W)])

    return embed_bag


def kernel(idx, embedding):
    idx2 = idx.reshape(B * L // IDX_PER_STEP, IDX_PER_STEP)
    out = _build()(idx2, embedding)
    return out.reshape(B, 1, D)


# speedup vs baseline: 1.0885x; 1.0885x over previous
import functools

import jax
import jax.numpy as jnp
from jax import lax
from jax.experimental import pallas as pl
from jax.experimental.pallas import tpu as pltpu
from jax.experimental.pallas import tpu_sc as plsc

B = 16384
L = 50
D = 32
NC = 2
NS = 16
NW = NC * NS
LANES = 16

BAGS_PER_STEP = 2
IDX_PER_STEP = BAGS_PER_STEP * L
BAGS_PER_W = B // NW
STEPS = BAGS_PER_W // BAGS_PER_STEP
NBUF = 4


@functools.cache
def _build():
    mesh = plsc.VectorSubcoreMesh(
        core_axis_name="c", subcore_axis_name="s", num_cores=NC, num_subcores=NS
    )

    @functools.partial(
        pl.kernel,
        out_type=jax.ShapeDtypeStruct((B, D), jnp.float32),
        mesh=mesh,
        compiler_params=pltpu.CompilerParams(use_tc_tiling_on_sc=False),
        scratch_types=[
            pltpu.VMEM((STEPS, IDX_PER_STEP), jnp.int32),
            pltpu.VMEM((NBUF, IDX_PER_STEP, D), jnp.float32),
            pltpu.VMEM((BAGS_PER_W, D), jnp.float32),
            pltpu.SemaphoreType.DMA((NBUF,)),
        ],
    )
    def embed_bag(idx_hbm, table_hbm, out_hbm, idx_v, rows_v, out_v, sems):
        wid = lax.axis_index("s") * NC + lax.axis_index("c")

        pltpu.sync_copy(idx_hbm.at[pl.ds(wid * STEPS, STEPS)], idx_v)

        def gather(step, slot):
            return pltpu.async_copy(
                table_hbm.at[idx_v.at[step]], rows_v.at[slot], sems.at[slot]
            )

        for b in range(NBUF):
            gather(b, b)

        inv = jnp.float32(1.0 / L)

        def outer(i, carry):
            gbase = i * NBUF
            for b in range(NBUF):
                g = gbase + b
                pltpu.make_async_copy(
                    table_hbm.at[idx_v.at[g]], rows_v.at[b], sems.at[b]
                ).wait()
                for bag in range(BAGS_PER_STEP):
                    r0 = bag * L
                    acc = [rows_v[b, r0 + j, pl.ds(h * LANES, LANES)]
                           for j in range(4) for h in range(2)]
                    for r in range(4, L):
                        acc[2 * (r % 4)] += rows_v[b, r0 + r, pl.ds(0, LANES)]
                        acc[2 * (r % 4) + 1] += rows_v[
                            b, r0 + r, pl.ds(LANES, LANES)]
                    out_row = g * BAGS_PER_STEP + bag
                    out_v[out_row, pl.ds(0, LANES)] = (
                        (acc[0] + acc[2]) + (acc[4] + acc[6])) * inv
                    out_v[out_row, pl.ds(LANES, LANES)] = (
                        (acc[1] + acc[3]) + (acc[5] + acc[7])) * inv

                @pl.when(g + NBUF < STEPS)
                def _():
                    gather(g + NBUF, b)

            return carry

        lax.fori_loop(0, STEPS // NBUF, outer, 0)

        pltpu.sync_copy(out_v, out_hbm.at[pl.ds(wid * BAGS_PER_W, BAGS_PER_W)])

    return embed_bag


def kernel(idx, embedding):
    idx2 = idx.reshape(B * L // IDX_PER_STEP, IDX_PER_STEP)
    out = _build()(idx2, embedding)
    return out.reshape(B, 1, D)
